# baseline (speedup 1.0000x reference)
import functools

import jax
import jax.numpy as jnp
from jax.experimental import pallas as pl
from jax.experimental.pallas import tpu as pltpu

BATCH = 32
T = 197
D = 256
S = 16
DEPTH = 5
FLAT = T * BATCH


LN2 = 0.6931471805599453


def _layernorm(x, w, b, eps=1e-5):
    mu = jnp.mean(x, axis=-1, keepdims=True)
    var = jnp.mean(x * x, axis=-1, keepdims=True) - mu * mu
    return (x - mu) / jnp.sqrt(var + eps) * w + b


def _net_kernel(
    x_ref,
    cls_ref,
    pos_ref,
    at_ref,
    lnw_ref,
    lnb_ref,
    wd_ref,
    bd_ref,
    wb_ref,
    wc_ref,
    dvec_ref,
    wo_ref,
    bo_ref,
    nw_ref,
    nb_ref,
    hw_ref,
    hb_ref,
    out_ref,
    h_ref,
    dd_ref,
    gg_ref,
    ys_ref,
    bt_ref,
    ct_ref,
    sta_ref,
    stb_ref,
):
    cls_b = jnp.broadcast_to(cls_ref[...], (1, BATCH, D))
    h0 = jnp.concatenate([cls_b, x_ref[...]], axis=0)
    h_ref[...] = h0 + pos_ref[...]

    for L in range(DEPTH):
        hv = h_ref[...]
        ln = _layernorm(hv, lnw_ref[L], lnb_ref[L])
        flat = ln.reshape(FLAT, D)
        z = jnp.dot(flat, wd_ref[L], preferred_element_type=jnp.float32) + bd_ref[L]
        u = jnp.log2(1.0 + jnp.exp(z))
        bm = jnp.dot(flat, wb_ref[L], preferred_element_type=jnp.float32)
        cm = jnp.dot(flat, wc_ref[L], preferred_element_type=jnp.float32)
        u3 = u.reshape(T, BATCH, D)
        dd_ref[0:T] = u3
        dd_ref[T] = jnp.zeros((BATCH, D), jnp.float32)
        gg_ref[0:T] = u3 * ln
        gg_ref[T] = jnp.zeros((BATCH, D), jnp.float32)
        bt_ref[0:T] = LN2 * jnp.transpose(bm.reshape(T, BATCH, S), (0, 2, 1))
        bt_ref[T] = jnp.zeros((S, BATCH), jnp.float32)
        ct_ref[0:T] = jnp.transpose(cm.reshape(T, BATCH, S), (0, 2, 1))
        ct_ref[T] = jnp.zeros((S, BATCH), jnp.float32)
        sta_ref[...] = jnp.zeros((S, BATCH, D), jnp.float32)

        def substep(t, src_ref, dst_ref):
            g = gg_ref[t]
            b_t = bt_ref[t][:, :, None]
            c_t = ct_ref[t][:, :, None]
            q1 = jnp.exp2(-dd_ref[t])
            q2 = q1 * q1
            q3 = q2 * q1
            q4 = q2 * q2
            q5 = q4 * q1
            q6 = q4 * q2
            q7 = q4 * q3
            q8 = q4 * q4
            q9 = q8 * q1
            q10 = q8 * q2
            q11 = q8 * q3
            q12 = q8 * q4
            q13 = q8 * q5
            q14 = q8 * q6
            q15 = q8 * q7
            q16 = q8 * q8
            da = jnp.stack(
                [q1, q2, q3, q4, q5, q6, q7, q8,
                 q9, q10, q11, q12, q13, q14, q15, q16], axis=0)
            st = da * src_ref[...] + b_t * g[None, :, :]
            dst_ref[...] = st
            ys_ref[t] = jnp.sum(c_t * st, axis=0)

        def step(i, _):
            t = 2 * i
            substep(t, sta_ref, stb_ref)
            substep(t + 1, stb_ref, sta_ref)
            return 0

        jax.lax.fori_loop(0, (T + 1) // 2, step, 0, unroll=4)

        y = ys_ref[0:T] + dvec_ref[L] * ln
        out = (
            jnp.dot(y.reshape(FLAT, D), wo_ref[L], preferred_element_type=jnp.float32)
            + bo_ref[L]
        )
        h_ref[...] = hv + out.reshape(T, BATCH, D)

    hv = h_ref[...]
    ln = _layernorm(hv, nw_ref[...], nb_ref[...])
    pooled = jnp.mean(ln[1:], axis=0)
    logits = (
        jnp.dot(pooled, hw_ref[...], preferred_element_type=jnp.float32)
        + hb_ref[...]
    )
    out_ref[...] = 1.0 / (1.0 + jnp.exp(-logits))


@jax.jit
def kernel(x, params, pos_embed):
    blocks = params['blocks']
    stack = lambda name: jnp.stack([b[name] for b in blocks])
    x_tm = jnp.transpose(x, (1, 0, 2))
    pos_tm = jnp.transpose(pos_embed, (1, 0, 2))
    at = jnp.transpose(-jnp.exp(stack('A_log')), (0, 2, 1))

    vmem = functools.partial(pl.BlockSpec, memory_space=pltpu.MemorySpace.VMEM)
    f32 = jnp.float32
    return pl.pallas_call(
        _net_kernel,
        out_shape=jax.ShapeDtypeStruct((BATCH, T - 1), f32),
        in_specs=[vmem()] * 17,
        out_specs=vmem(),
        scratch_shapes=[
            pltpu.VMEM((T, BATCH, D), f32),
            pltpu.VMEM((T + 1, BATCH, D), f32),
            pltpu.VMEM((T + 1, BATCH, D), f32),
            pltpu.VMEM((T + 1, BATCH, D), f32),
            pltpu.VMEM((T + 1, S, BATCH), f32),
            pltpu.VMEM((T + 1, S, BATCH), f32),
            pltpu.VMEM((S, BATCH, D), f32),
            pltpu.VMEM((S, BATCH, D), f32),
        ],
    )(
        x_tm,
        params['cls_token'],
        pos_tm,
        at,
        stack('ln_w'),
        stack('ln_b'),
        stack('W_delta'),
        stack('b_delta'),
        stack('W_B'),
        stack('W_C'),
        stack('D'),
        stack('W_out'),
        stack('b_out'),
        params['norm_w'],
        params['norm_b'],
        params['head_W'],
        params['head_b'],
    )


# speedup vs baseline: 1.0153x; 1.0153x over previous
import functools

import jax
import jax.numpy as jnp
from jax.experimental import pallas as pl
from jax.experimental.pallas import tpu as pltpu

BATCH = 32
T = 197
D = 256
S = 16
DEPTH = 5
FLAT = T * BATCH


LN2 = 0.6931471805599453


def _layernorm(x, w, b, eps=1e-5):
    mu = jnp.mean(x, axis=-1, keepdims=True)
    var = jnp.mean(x * x, axis=-1, keepdims=True) - mu * mu
    return (x - mu) / jnp.sqrt(var + eps) * w + b


def _net_kernel(
    x_ref,
    cls_ref,
    pos_ref,
    at_ref,
    lnw_ref,
    lnb_ref,
    wd_ref,
    bd_ref,
    wb_ref,
    wc_ref,
    dvec_ref,
    wo_ref,
    bo_ref,
    nw_ref,
    nb_ref,
    hw_ref,
    hb_ref,
    out_ref,
    h_ref,
    dd_ref,
    gg_ref,
    ys_ref,
    bt_ref,
    ct_ref,
    sta_ref,
    stb_ref,
):
    cls_b = jnp.broadcast_to(cls_ref[...], (1, BATCH, D))
    h0 = jnp.concatenate([cls_b, x_ref[...]], axis=0)
    h_ref[...] = h0 + pos_ref[...]

    for L in range(DEPTH):
        hv = h_ref[...]
        ln = _layernorm(hv, lnw_ref[L], lnb_ref[L])
        flat = ln.reshape(FLAT, D)
        z = jnp.dot(flat, wd_ref[L], preferred_element_type=jnp.float32, precision=jax.lax.Precision.DEFAULT) + bd_ref[L]
        u = jnp.log2(1.0 + jnp.exp(z))
        bm = jnp.dot(flat, wb_ref[L], preferred_element_type=jnp.float32, precision=jax.lax.Precision.DEFAULT)
        cm = jnp.dot(flat, wc_ref[L], preferred_element_type=jnp.float32, precision=jax.lax.Precision.DEFAULT)
        u3 = u.reshape(T, BATCH, D)
        dd_ref[0:T] = u3
        dd_ref[T] = jnp.zeros((BATCH, D), jnp.float32)
        gg_ref[0:T] = u3 * ln
        gg_ref[T] = jnp.zeros((BATCH, D), jnp.float32)
        bt_ref[0:T] = LN2 * jnp.transpose(bm.reshape(T, BATCH, S), (0, 2, 1))
        bt_ref[T] = jnp.zeros((S, BATCH), jnp.float32)
        ct_ref[0:T] = jnp.transpose(cm.reshape(T, BATCH, S), (0, 2, 1))
        ct_ref[T] = jnp.zeros((S, BATCH), jnp.float32)
        sta_ref[...] = jnp.zeros((S, BATCH, D), jnp.float32)
        a3 = at_ref[L][:, None, :]

        def substep(t, src_ref, dst_ref):
            dt = dd_ref[t]
            g = gg_ref[t]
            b_t = bt_ref[t][:, :, None]
            c_t = ct_ref[t][:, :, None]
            da = jnp.exp2(dt[None, :, :] * a3)
            st = da * src_ref[...] + b_t * g[None, :, :]
            dst_ref[...] = st
            ys_ref[t] = jnp.sum(c_t * st, axis=0)

        def step(i, _):
            t = 2 * i
            substep(t, sta_ref, stb_ref)
            substep(t + 1, stb_ref, sta_ref)
            return 0

        jax.lax.fori_loop(0, (T + 1) // 2, step, 0, unroll=4)

        y = ys_ref[0:T] + dvec_ref[L] * ln
        out = (
            jnp.dot(y.reshape(FLAT, D), wo_ref[L], preferred_element_type=jnp.float32, precision=jax.lax.Precision.DEFAULT)
            + bo_ref[L]
        )
        h_ref[...] = hv + out.reshape(T, BATCH, D)

    hv = h_ref[...]
    ln = _layernorm(hv, nw_ref[...], nb_ref[...])
    pooled = jnp.mean(ln[1:], axis=0)
    logits = (
        jnp.dot(pooled, hw_ref[...], preferred_element_type=jnp.float32, precision=jax.lax.Precision.DEFAULT)
        + hb_ref[...]
    )
    out_ref[...] = 1.0 / (1.0 + jnp.exp(-logits))


@jax.jit
def kernel(x, params, pos_embed):
    blocks = params['blocks']
    stack = lambda name: jnp.stack([b[name] for b in blocks])
    x_tm = jnp.transpose(x, (1, 0, 2))
    pos_tm = jnp.transpose(pos_embed, (1, 0, 2))
    at = jnp.transpose(-jnp.exp(stack('A_log')), (0, 2, 1))

    vmem = functools.partial(pl.BlockSpec, memory_space=pltpu.MemorySpace.VMEM)
    f32 = jnp.float32
    return pl.pallas_call(
        _net_kernel,
        out_shape=jax.ShapeDtypeStruct((BATCH, T - 1), f32),
        in_specs=[vmem()] * 17,
        out_specs=vmem(),
        scratch_shapes=[
            pltpu.VMEM((T, BATCH, D), f32),
            pltpu.VMEM((T + 1, BATCH, D), f32),
            pltpu.VMEM((T + 1, BATCH, D), f32),
            pltpu.VMEM((T + 1, BATCH, D), f32),
            pltpu.VMEM((T + 1, S, BATCH), f32),
            pltpu.VMEM((T + 1, S, BATCH), f32),
            pltpu.VMEM((S, BATCH, D), f32),
            pltpu.VMEM((S, BATCH, D), f32),
        ],
    )(
        x_tm,
        params['cls_token'],
        pos_tm,
        at,
        stack('ln_w'),
        stack('ln_b'),
        stack('W_delta'),
        stack('b_delta'),
        stack('W_B'),
        stack('W_C'),
        stack('D'),
        stack('W_out'),
        stack('b_out'),
        params['norm_w'],
        params['norm_b'],
        params['head_W'],
        params['head_b'],
    )


# speedup vs baseline: 1.0411x; 1.0254x over previous
import functools

import jax
import jax.numpy as jnp
from jax.experimental import pallas as pl
from jax.experimental.pallas import tpu as pltpu

BATCH = 32
T = 197
D = 256
S = 16
DEPTH = 5
FLAT = T * BATCH


LN2 = 0.6931471805599453


def _layernorm(x, w, b, eps=1e-5):
    mu = jnp.mean(x, axis=-1, keepdims=True)
    var = jnp.mean(x * x, axis=-1, keepdims=True) - mu * mu
    return (x - mu) / jnp.sqrt(var + eps) * w + b


def _net_kernel(
    x_ref,
    cls_ref,
    pos_ref,
    at_ref,
    lnw_ref,
    lnb_ref,
    wd_ref,
    bd_ref,
    wb_ref,
    wc_ref,
    dvec_ref,
    wo_ref,
    bo_ref,
    nw_ref,
    nb_ref,
    hw_ref,
    hb_ref,
    out_ref,
    h_ref,
    dd_ref,
    gg_ref,
    ys_ref,
    bt_ref,
    ct_ref,
    sta_ref,
    stb_ref,
):
    cls_b = jnp.broadcast_to(cls_ref[...], (1, BATCH, D))
    h0 = jnp.concatenate([cls_b, x_ref[...]], axis=0)
    h_ref[...] = h0 + pos_ref[...]

    for L in range(DEPTH):
        hv = h_ref[...]
        ln = _layernorm(hv, lnw_ref[L], lnb_ref[L])
        flat = ln.reshape(FLAT, D)
        z = jnp.dot(flat, wd_ref[L], preferred_element_type=jnp.float32, precision=jax.lax.Precision.DEFAULT) + bd_ref[L]
        u = jnp.log2(1.0 + jnp.exp(z))
        bm = jnp.dot(flat, wb_ref[L], preferred_element_type=jnp.float32, precision=jax.lax.Precision.DEFAULT)
        cm = jnp.dot(flat, wc_ref[L], preferred_element_type=jnp.float32, precision=jax.lax.Precision.DEFAULT)
        u3 = u.reshape(T, BATCH, D)
        dd_ref[0:T] = u3
        dd_ref[T] = jnp.zeros((BATCH, D), jnp.float32)
        gg_ref[0:T] = u3 * ln
        gg_ref[T] = jnp.zeros((BATCH, D), jnp.float32)
        bt_ref[0:T] = LN2 * jnp.transpose(bm.reshape(T, BATCH, S), (0, 2, 1))
        bt_ref[T] = jnp.zeros((S, BATCH), jnp.float32)
        ct_ref[0:T] = jnp.transpose(cm.reshape(T, BATCH, S), (0, 2, 1))
        ct_ref[T] = jnp.zeros((S, BATCH), jnp.float32)
        sta_ref[...] = jnp.zeros((S, BATCH, D), jnp.float32)
        a3 = at_ref[L][:, None, :]

        def substep(t, src_ref, dst_ref):
            dt = dd_ref[t]
            g = gg_ref[t]
            b_t = bt_ref[t][:, :, None]
            c_t = ct_ref[t][:, :, None]
            da = jnp.exp2(dt[None, :, :] * a3)
            st = da * src_ref[...] + b_t * g[None, :, :]
            dst_ref[...] = st
            ys_ref[t] = jnp.sum(c_t * st, axis=0)

        def step(i, _):
            t = 2 * i
            substep(t, sta_ref, stb_ref)
            substep(t + 1, stb_ref, sta_ref)
            return 0

        jax.lax.fori_loop(0, (T + 1) // 2, step, 0, unroll=8)

        y = ys_ref[0:T] + dvec_ref[L] * ln
        out = (
            jnp.dot(y.reshape(FLAT, D), wo_ref[L], preferred_element_type=jnp.float32, precision=jax.lax.Precision.DEFAULT)
            + bo_ref[L]
        )
        h_ref[...] = hv + out.reshape(T, BATCH, D)

    hv = h_ref[...]
    ln = _layernorm(hv, nw_ref[...], nb_ref[...])
    pooled = jnp.mean(ln[1:], axis=0)
    logits = (
        jnp.dot(pooled, hw_ref[...], preferred_element_type=jnp.float32, precision=jax.lax.Precision.DEFAULT)
        + hb_ref[...]
    )
    out_ref[...] = 1.0 / (1.0 + jnp.exp(-logits))


@jax.jit
def kernel(x, params, pos_embed):
    blocks = params['blocks']
    stack = lambda name: jnp.stack([b[name] for b in blocks])
    x_tm = jnp.transpose(x, (1, 0, 2))
    pos_tm = jnp.transpose(pos_embed, (1, 0, 2))
    at = jnp.transpose(-jnp.exp(stack('A_log')), (0, 2, 1))

    vmem = functools.partial(pl.BlockSpec, memory_space=pltpu.MemorySpace.VMEM)
    f32 = jnp.float32
    return pl.pallas_call(
        _net_kernel,
        out_shape=jax.ShapeDtypeStruct((BATCH, T - 1), f32),
        in_specs=[vmem()] * 17,
        out_specs=vmem(),
        scratch_shapes=[
            pltpu.VMEM((T, BATCH, D), f32),
            pltpu.VMEM((T + 1, BATCH, D), f32),
            pltpu.VMEM((T + 1, BATCH, D), f32),
            pltpu.VMEM((T + 1, BATCH, D), f32),
            pltpu.VMEM((T + 1, S, BATCH), f32),
            pltpu.VMEM((T + 1, S, BATCH), f32),
            pltpu.VMEM((S, BATCH, D), f32),
            pltpu.VMEM((S, BATCH, D), f32),
        ],
    )(
        x_tm,
        params['cls_token'],
        pos_tm,
        at,
        stack('ln_w'),
        stack('ln_b'),
        stack('W_delta'),
        stack('b_delta'),
        stack('W_B'),
        stack('W_C'),
        stack('D'),
        stack('W_out'),
        stack('b_out'),
        params['norm_w'],
        params['norm_b'],
        params['head_W'],
        params['head_b'],
    )


# speedup vs baseline: 1.0494x; 1.0080x over previous
import functools

import jax
import jax.numpy as jnp
from jax.experimental import pallas as pl
from jax.experimental.pallas import tpu as pltpu

BATCH = 32
T = 197
D = 256
S = 16
DEPTH = 5
FLAT = T * BATCH


LN2 = 0.6931471805599453


def _layernorm(x, w, b, eps=1e-5):
    mu = jnp.mean(x, axis=-1, keepdims=True)
    var = jnp.mean(x * x, axis=-1, keepdims=True) - mu * mu
    return (x - mu) / jnp.sqrt(var + eps) * w + b


def _net_kernel(
    x_ref,
    cls_ref,
    pos_ref,
    at_ref,
    lnw_ref,
    lnb_ref,
    wd_ref,
    bd_ref,
    wb_ref,
    wc_ref,
    dvec_ref,
    wo_ref,
    bo_ref,
    nw_ref,
    nb_ref,
    hw_ref,
    hb_ref,
    out_ref,
    h_ref,
    dd_ref,
    gg_ref,
    ys_ref,
    bt_ref,
    ct_ref,
    sta_ref,
    stb_ref,
):
    cls_b = jnp.broadcast_to(cls_ref[...], (1, BATCH, D))
    h0 = jnp.concatenate([cls_b, x_ref[...]], axis=0)
    h_ref[...] = h0 + pos_ref[...]

    for L in range(DEPTH):
        hv = h_ref[...]
        ln = _layernorm(hv, lnw_ref[L], lnb_ref[L])
        flat = ln.reshape(FLAT, D)
        z = jnp.dot(flat, wd_ref[L], preferred_element_type=jnp.float32, precision=jax.lax.Precision.DEFAULT) + bd_ref[L]
        u = jnp.log2(1.0 + jnp.exp(z))
        bm = jnp.dot(flat, wb_ref[L], preferred_element_type=jnp.float32, precision=jax.lax.Precision.DEFAULT)
        cm = jnp.dot(flat, wc_ref[L], preferred_element_type=jnp.float32, precision=jax.lax.Precision.DEFAULT)
        u3 = u.reshape(T, BATCH, D)
        dd_ref[0:T] = u3
        dd_ref[T] = jnp.zeros((BATCH, D), jnp.float32)
        gg_ref[0:T] = u3 * ln
        gg_ref[T] = jnp.zeros((BATCH, D), jnp.float32)
        bt_ref[0:T] = LN2 * jnp.transpose(bm.reshape(T, BATCH, S), (0, 2, 1))
        bt_ref[T] = jnp.zeros((S, BATCH), jnp.float32)
        ct_ref[0:T] = jnp.transpose(cm.reshape(T, BATCH, S), (0, 2, 1))
        ct_ref[T] = jnp.zeros((S, BATCH), jnp.float32)
        sta_ref[...] = jnp.zeros((S, BATCH, D), jnp.float32)
        a3 = at_ref[L][:, None, :]

        def substep(t, src_ref, dst_ref):
            dt = dd_ref[t]
            g = gg_ref[t]
            b_t = bt_ref[t][:, :, None]
            c_t = ct_ref[t][:, :, None]
            da = jnp.exp2(dt[None, :, :] * a3)
            st = da * src_ref[...] + b_t * g[None, :, :]
            dst_ref[...] = st
            ys_ref[t] = jnp.sum(c_t * st, axis=0)

        def step(i, _):
            t = 2 * i
            substep(t, sta_ref, stb_ref)
            substep(t + 1, stb_ref, sta_ref)
            return 0

        jax.lax.fori_loop(0, (T + 1) // 2, step, 0, unroll=16)

        y = ys_ref[0:T] + dvec_ref[L] * ln
        out = (
            jnp.dot(y.reshape(FLAT, D), wo_ref[L], preferred_element_type=jnp.float32, precision=jax.lax.Precision.DEFAULT)
            + bo_ref[L]
        )
        h_ref[...] = hv + out.reshape(T, BATCH, D)

    hv = h_ref[...]
    ln = _layernorm(hv, nw_ref[...], nb_ref[...])
    pooled = jnp.mean(ln[1:], axis=0)
    logits = (
        jnp.dot(pooled, hw_ref[...], preferred_element_type=jnp.float32, precision=jax.lax.Precision.DEFAULT)
        + hb_ref[...]
    )
    out_ref[...] = 1.0 / (1.0 + jnp.exp(-logits))


@jax.jit
def kernel(x, params, pos_embed):
    blocks = params['blocks']
    stack = lambda name: jnp.stack([b[name] for b in blocks])
    x_tm = jnp.transpose(x, (1, 0, 2))
    pos_tm = jnp.transpose(pos_embed, (1, 0, 2))
    at = jnp.transpose(-jnp.exp(stack('A_log')), (0, 2, 1))

    vmem = functools.partial(pl.BlockSpec, memory_space=pltpu.MemorySpace.VMEM)
    f32 = jnp.float32
    return pl.pallas_call(
        _net_kernel,
        out_shape=jax.ShapeDtypeStruct((BATCH, T - 1), f32),
        in_specs=[vmem()] * 17,
        out_specs=vmem(),
        scratch_shapes=[
            pltpu.VMEM((T, BATCH, D), f32),
            pltpu.VMEM((T + 1, BATCH, D), f32),
            pltpu.VMEM((T + 1, BATCH, D), f32),
            pltpu.VMEM((T + 1, BATCH, D), f32),
            pltpu.VMEM((T + 1, S, BATCH), f32),
            pltpu.VMEM((T + 1, S, BATCH), f32),
            pltpu.VMEM((S, BATCH, D), f32),
            pltpu.VMEM((S, BATCH, D), f32),
        ],
    )(
        x_tm,
        params['cls_token'],
        pos_tm,
        at,
        stack('ln_w'),
        stack('ln_b'),
        stack('W_delta'),
        stack('b_delta'),
        stack('W_B'),
        stack('W_C'),
        stack('D'),
        stack('W_out'),
        stack('b_out'),
        params['norm_w'],
        params['norm_b'],
        params['head_W'],
        params['head_b'],
    )
